# baseline (speedup 1.0000x reference)
import functools

import jax
import jax.numpy as jnp
from jax.experimental import pallas as pl
from jax.experimental.pallas import tpu as pltpu

_TAU = 1.0
_C = 14
_LOSS_WEIGHT = 1.0


def _fold(x):
    h, w = x.shape
    while w > 128:
        w //= 2
        x = x[:, :w] + x[:, w:]
    while h > 8:
        h //= 2
        x = x[:h, :] + x[h:, :]
    return x


def _kl_loss_kernel(gt_ref, s_ref, t_ref, out_ref, acc_s, acc_c, *, n_hblk, n_b):
    b = pl.program_id(0)
    ih = pl.program_id(1)

    @pl.when(jnp.logical_and(b == 0, ih == 0))
    def _init():
        acc_s[...] = jnp.zeros_like(acc_s)
        acc_c[...] = jnp.zeros_like(acc_c)

    se = None
    for c in range(_C):
        sc = s_ref[0, c]
        tc = t_ref[0, c]
        if _TAU != 1.0:
            sc = sc / _TAU
            tc = tc / _TAU
        esc = jnp.exp(sc)
        etc = jnp.exp(tc)
        wc = etc * (tc - sc)
        if se is None:
            se, te, we = esc, etc, wc
        else:
            se = se + esc
            te = te + etc
            we = we + wc

    kl = we / te + jnp.log(se) - jnp.log(te)

    gt = gt_ref[0, 0]
    ones = jnp.ones_like(kl)
    for c in range(1, _C):
        m = gt == c
        fk = _fold(jnp.where(m, kl, 0.0))
        fc = _fold(jnp.where(m, ones, 0.0))
        row = pl.ds(b * 128 + c * 8, 8)
        acc_s[row, :] += fk
        acc_c[row, :] += fc

    @pl.when(jnp.logical_and(b == n_b - 1, ih == n_hblk - 1))
    def _finish():
        sums3 = acc_s[...].reshape(64, 8, 128)
        cnts3 = acc_c[...].reshape(64, 8, 128)
        sums = jnp.sum(jnp.sum(sums3, axis=1), axis=1, keepdims=True)
        cnts = jnp.sum(jnp.sum(cnts3, axis=1), axis=1, keepdims=True)
        rid = jax.lax.broadcasted_iota(jnp.int32, sums.shape, 0)
        cid = jax.lax.bitwise_and(rid, 15)
        valid = jnp.logical_and(cid >= 1, cid <= _C - 1)
        valid = jnp.logical_and(valid, cnts > 0.0)
        per = jnp.where(valid, sums / (_C * jnp.maximum(cnts, 1.0)), 0.0)
        out_ref[...] = jnp.sum(per, axis=0, keepdims=True) * (_TAU ** 2) * _LOSS_WEIGHT


def kernel(preds_S, preds_T, gt_labels):
    B, C, H, W = preds_S.shape
    BLK_H = 256
    n_hblk = H // BLK_H

    gt = gt_labels.astype(jnp.int32)

    out = pl.pallas_call(
        functools.partial(_kl_loss_kernel, n_hblk=n_hblk, n_b=B),
        grid=(B, n_hblk),
        in_specs=[
            pl.BlockSpec((1, 1, BLK_H, W), lambda b, ih: (b, 0, ih, 0)),
            pl.BlockSpec((1, C, BLK_H, W), lambda b, ih: (b, 0, ih, 0)),
            pl.BlockSpec((1, C, BLK_H, W), lambda b, ih: (b, 0, ih, 0)),
        ],
        out_specs=pl.BlockSpec((1, 1), lambda b, ih: (0, 0)),
        out_shape=jax.ShapeDtypeStruct((1, 1), jnp.float32),
        scratch_shapes=[
            pltpu.VMEM((512, 128), jnp.float32),
            pltpu.VMEM((512, 128), jnp.float32),
        ],
        compiler_params=pltpu.CompilerParams(
            dimension_semantics=("arbitrary", "arbitrary"),
        ),
    )(gt, preds_S, preds_T)
    return out[0, 0]
